# Initial kernel scaffold; baseline (speedup 1.0000x reference)
import jax
import jax.numpy as jnp
from jax.experimental import pallas as pl


def kernel(dec_h0, dec_c0, enc_outputs, enc_input, emb, Wx, Wh, b, W1, b1, W2, b2, v, bv):
    raise NotImplementedError("write your pallas kernel here")


# speedup vs baseline: 2.1956x; 2.1956x over previous
import jax
import jax.numpy as jnp
from jax.experimental import pallas as pl
from jax.experimental.pallas import tpu as pltpu

_BIG_NUMBER = 1.0e6
_SOS_CODE = 1


def _decoder_body(eo_ref, einp_ref, h0_ref, c0_ref, emb_ref, wx_ref, wh_ref,
                  b_ref, w1_ref, b1_ref, w2_ref, b2_ref, v_ref, bv_ref,
                  out_ref, ep_scr, h_scr, c_scr, mask_scr, x_scr):
    Bb, L, D = ep_scr.shape
    E = x_scr.shape[1]
    V = emb_ref.shape[0]
    t = pl.program_id(1)

    @pl.when(t == 0)
    def _init():
        for bb in range(0, Bb, 8):
            eo2 = eo_ref[bb:bb + 8].reshape(8 * L, D)
            ep = jnp.dot(eo2, w1_ref[...], preferred_element_type=jnp.float32)
            ep_scr[bb:bb + 8] = (ep + b1_ref[...]).reshape(8, L, D)
        h_scr[...] = h0_ref[...]
        c_scr[...] = c0_ref[...]
        mask_scr[...] = jnp.zeros((Bb, L), jnp.float32)
        x_scr[...] = jnp.broadcast_to(emb_ref[_SOS_CODE][None, :], (Bb, E))

    h = h_scr[...]
    c = c_scr[...]
    x16 = x_scr[...]

    z = (jnp.dot(x16, wx_ref[...], preferred_element_type=jnp.float32)
         + jnp.dot(h.astype(jnp.bfloat16), wh_ref[...],
                   preferred_element_type=jnp.float32)
         + b_ref[...])
    gi = jax.nn.sigmoid(z[:, :D])
    gf = jax.nn.sigmoid(z[:, D:2 * D])
    gg = z[:, 2 * D:3 * D]
    go = jax.nn.sigmoid(z[:, 3 * D:])
    c_new = gf * c + gi * jnp.tanh(gg)
    h_new = go * jnp.tanh(c_new)
    h_scr[...] = h_new
    c_scr[...] = c_new

    h16 = h_new.astype(jnp.bfloat16)
    q = (jnp.dot(h16, w2_ref[...], preferred_element_type=jnp.float32)
         + b2_ref[...])

    v8 = jnp.broadcast_to(v_ref[...], (8, D))
    parts = []
    for bb in range(0, Bb, 8):
        s_c = jnp.tanh(ep_scr[bb:bb + 8] + q[bb:bb + 8][:, None, :])
        s_rc = s_c.astype(jnp.bfloat16).astype(jnp.float32).reshape(8 * L, D)
        lt = jax.lax.dot_general(v8, s_rc, (((1,), (1,)), ((), ())),
                                 preferred_element_type=jnp.float32)
        parts.extend(lt[j:j + 1, j * L:(j + 1) * L] for j in range(8))
    logits = jnp.concatenate(parts, axis=0) + bv_ref[0, 0]
    logits = logits - mask_scr[...] * _BIG_NUMBER

    m = jnp.max(logits, axis=-1, keepdims=True)
    e = jnp.exp(logits - m)
    p = e / jnp.sum(e, axis=-1, keepdims=True)
    out_ref[...] = p.reshape(1, Bb, L)

    iota_l = jax.lax.broadcasted_iota(jnp.int32, (Bb, L), 1)
    p_max = jnp.max(p, axis=-1, keepdims=True)
    idx = jnp.min(jnp.where(p == p_max, iota_l, L), axis=-1, keepdims=True)
    ohf = jnp.where(iota_l == idx, 1.0, 0.0)
    mask_scr[...] = mask_scr[...] + ohf
    tokf = jnp.sum(ohf * einp_ref[...], axis=-1, keepdims=True)
    iota_v = jax.lax.broadcasted_iota(jnp.int32, (Bb, V), 1)
    ohv = jnp.where(iota_v == tokf.astype(jnp.int32),
                    1.0, 0.0).astype(jnp.bfloat16)
    x_next = jnp.dot(ohv, emb_ref[...], preferred_element_type=jnp.float32)
    x_scr[...] = x_next.astype(jnp.bfloat16)


def kernel(dec_h0, dec_c0, enc_outputs, enc_input, emb, Wx, Wh, b,
           W1, b1, W2, b2, v, bv, *, interpret=False):
    B, L, D = enc_outputs.shape
    V, E = emb.shape
    Bb = 64
    NB = B // Bb

    bf = jnp.bfloat16
    eo16 = enc_outputs.astype(bf)
    einpf = enc_input.astype(jnp.float32)
    emb16 = emb.astype(bf)
    wx16 = Wx.astype(bf)
    wh16 = Wh.astype(bf)
    w116 = W1.astype(bf)
    w216 = W2.astype(bf)
    b2d = b.reshape(1, 4 * D)
    b12d = b1.reshape(1, D)
    b22d = b2.reshape(1, D)
    v2d = v.reshape(1, D)
    bv2d = bv.reshape(1, 1)

    fixed = lambda i, t: (0, 0)
    grid = (NB, L)
    out = pl.pallas_call(
        _decoder_body,
        grid=grid,
        in_specs=[
            pl.BlockSpec((Bb, L, D), lambda i, t: (i, 0, 0)),
            pl.BlockSpec((Bb, L), lambda i, t: (i, 0)),
            pl.BlockSpec((Bb, D), lambda i, t: (i, 0)),
            pl.BlockSpec((Bb, D), lambda i, t: (i, 0)),
            pl.BlockSpec((V, E), fixed),
            pl.BlockSpec((E, 4 * D), fixed),
            pl.BlockSpec((D, 4 * D), fixed),
            pl.BlockSpec((1, 4 * D), fixed),
            pl.BlockSpec((D, D), fixed),
            pl.BlockSpec((1, D), fixed),
            pl.BlockSpec((D, D), fixed),
            pl.BlockSpec((1, D), fixed),
            pl.BlockSpec((1, D), fixed),
            pl.BlockSpec((1, 1), fixed),
        ],
        out_specs=pl.BlockSpec((1, Bb, L), lambda i, t: (t, i, 0)),
        out_shape=jax.ShapeDtypeStruct((L, B, L), jnp.float32),
        scratch_shapes=[
            pltpu.VMEM((Bb, L, D), jnp.float32),
            pltpu.VMEM((Bb, D), jnp.float32),
            pltpu.VMEM((Bb, D), jnp.float32),
            pltpu.VMEM((Bb, L), jnp.float32),
            pltpu.VMEM((Bb, E), jnp.bfloat16),
        ],
        compiler_params=pltpu.CompilerParams(
            dimension_semantics=("parallel", "arbitrary"),
            vmem_limit_bytes=57 * 1024 * 1024,
        ),
        name="ptr_decoder",
        interpret=interpret,
    )(eo16, einpf, dec_h0, dec_c0, emb16, wx16, wh16, b2d, w116, b12d,
      w216, b22d, v2d, bv2d)
    return jnp.transpose(out, (1, 0, 2))


# speedup vs baseline: 3.4822x; 1.5860x over previous
import numpy as np

import jax
import jax.numpy as jnp
from jax.experimental import pallas as pl
from jax.experimental.pallas import tpu as pltpu
from jax.sharding import Mesh, PartitionSpec as P

_BIG_NUMBER = 1.0e6
_SOS_CODE = 1


def _decoder_body(eo_ref, einp_ref, h0_ref, c0_ref, emb_ref, wx_ref, wh_ref,
                  b_ref, w1_ref, b1_ref, w2_ref, b2_ref, v_ref, bv_ref,
                  out_ref, ep_scr, h_scr, c_scr, mask_scr, x_scr):
    Bb, L, D = ep_scr.shape
    E = x_scr.shape[1]
    V = emb_ref.shape[0]
    t = pl.program_id(1)

    @pl.when(t == 0)
    def _init():
        for bb in range(0, Bb, 8):
            eo2 = eo_ref[bb:bb + 8].reshape(8 * L, D)
            ep = jnp.dot(eo2, w1_ref[...], preferred_element_type=jnp.float32)
            ep_scr[bb:bb + 8] = (ep + b1_ref[...]).reshape(8, L, D)
        h_scr[...] = h0_ref[...]
        c_scr[...] = c0_ref[...]
        mask_scr[...] = jnp.zeros((Bb, L), jnp.float32)
        x_scr[...] = jnp.broadcast_to(emb_ref[_SOS_CODE][None, :], (Bb, E))

    h = h_scr[...]
    c = c_scr[...]
    x16 = x_scr[...]

    z = (jnp.dot(x16, wx_ref[...], preferred_element_type=jnp.float32)
         + jnp.dot(h.astype(jnp.bfloat16), wh_ref[...],
                   preferred_element_type=jnp.float32)
         + b_ref[...])
    gi = jax.nn.sigmoid(z[:, :D])
    gf = jax.nn.sigmoid(z[:, D:2 * D])
    gg = z[:, 2 * D:3 * D]
    go = jax.nn.sigmoid(z[:, 3 * D:])
    c_new = gf * c + gi * jnp.tanh(gg)
    h_new = go * jnp.tanh(c_new)
    h_scr[...] = h_new
    c_scr[...] = c_new

    h16 = h_new.astype(jnp.bfloat16)
    q = (jnp.dot(h16, w2_ref[...], preferred_element_type=jnp.float32)
         + b2_ref[...])

    v8 = jnp.broadcast_to(v_ref[...], (8, D))
    parts = []
    for bb in range(0, Bb, 8):
        s_c = jnp.tanh(ep_scr[bb:bb + 8] + q[bb:bb + 8][:, None, :])
        s_rc = s_c.astype(jnp.bfloat16).astype(jnp.float32).reshape(8 * L, D)
        lt = jax.lax.dot_general(v8, s_rc, (((1,), (1,)), ((), ())),
                                 preferred_element_type=jnp.float32)
        parts.extend(lt[j:j + 1, j * L:(j + 1) * L] for j in range(8))
    logits = jnp.concatenate(parts, axis=0) + bv_ref[0, 0]
    logits = logits - mask_scr[...] * _BIG_NUMBER

    m = jnp.max(logits, axis=-1, keepdims=True)
    e = jnp.exp(logits - m)
    p = e / jnp.sum(e, axis=-1, keepdims=True)
    out_ref[...] = p.reshape(1, Bb, L)

    iota_l = jax.lax.broadcasted_iota(jnp.int32, (Bb, L), 1)
    p_max = jnp.max(p, axis=-1, keepdims=True)
    idx = jnp.min(jnp.where(p == p_max, iota_l, L), axis=-1, keepdims=True)
    ohf = jnp.where(iota_l == idx, 1.0, 0.0)
    mask_scr[...] = mask_scr[...] + ohf
    tokf = jnp.sum(ohf * einp_ref[...], axis=-1, keepdims=True)
    iota_v = jax.lax.broadcasted_iota(jnp.int32, (Bb, V), 1)
    ohv = jnp.where(iota_v == tokf.astype(jnp.int32),
                    1.0, 0.0).astype(jnp.bfloat16)
    x_next = jnp.dot(ohv, emb_ref[...], preferred_element_type=jnp.float32)
    x_scr[...] = x_next.astype(jnp.bfloat16)


def _decode_block(eo16, einpf, dec_h0, dec_c0, emb16, wx16, wh16, b2d,
                  w116, b12d, w216, b22d, v2d, bv2d, *, interpret=False):
    B, L, D = eo16.shape
    V, E = emb16.shape
    Bb = 64
    NB = B // Bb

    fixed = lambda i, t: (0, 0)
    grid = (NB, L)
    out = pl.pallas_call(
        _decoder_body,
        grid=grid,
        in_specs=[
            pl.BlockSpec((Bb, L, D), lambda i, t: (i, 0, 0)),
            pl.BlockSpec((Bb, L), lambda i, t: (i, 0)),
            pl.BlockSpec((Bb, D), lambda i, t: (i, 0)),
            pl.BlockSpec((Bb, D), lambda i, t: (i, 0)),
            pl.BlockSpec((V, E), fixed),
            pl.BlockSpec((E, 4 * D), fixed),
            pl.BlockSpec((D, 4 * D), fixed),
            pl.BlockSpec((1, 4 * D), fixed),
            pl.BlockSpec((D, D), fixed),
            pl.BlockSpec((1, D), fixed),
            pl.BlockSpec((D, D), fixed),
            pl.BlockSpec((1, D), fixed),
            pl.BlockSpec((1, D), fixed),
            pl.BlockSpec((1, 1), fixed),
        ],
        out_specs=pl.BlockSpec((1, Bb, L), lambda i, t: (t, i, 0)),
        out_shape=jax.ShapeDtypeStruct((L, B, L), jnp.float32),
        scratch_shapes=[
            pltpu.VMEM((Bb, L, D), jnp.float32),
            pltpu.VMEM((Bb, D), jnp.float32),
            pltpu.VMEM((Bb, D), jnp.float32),
            pltpu.VMEM((Bb, L), jnp.float32),
            pltpu.VMEM((Bb, E), jnp.bfloat16),
        ],
        compiler_params=pltpu.CompilerParams(
            dimension_semantics=("parallel", "arbitrary"),
            vmem_limit_bytes=57 * 1024 * 1024,
        ),
        name="ptr_decoder",
        interpret=interpret,
    )(eo16, einpf, dec_h0, dec_c0, emb16, wx16, wh16, b2d, w116, b12d,
      w216, b22d, v2d, bv2d)
    return out


def kernel(dec_h0, dec_c0, enc_outputs, enc_input, emb, Wx, Wh, b,
           W1, b1, W2, b2, v, bv, *, interpret=False):
    B, L, D = enc_outputs.shape
    V, E = emb.shape

    bf = jnp.bfloat16
    eo16 = enc_outputs.astype(bf)
    einpf = enc_input.astype(jnp.float32)
    emb16 = emb.astype(bf)
    wx16 = Wx.astype(bf)
    wh16 = Wh.astype(bf)
    w116 = W1.astype(bf)
    w216 = W2.astype(bf)
    b2d = b.reshape(1, 4 * D)
    b12d = b1.reshape(1, D)
    b22d = b2.reshape(1, D)
    v2d = v.reshape(1, D)
    bv2d = bv.reshape(1, 1)

    args = (eo16, einpf, dec_h0, dec_c0, emb16, wx16, wh16, b2d,
            w116, b12d, w216, b22d, v2d, bv2d)
    block = lambda *a: _decode_block(*a, interpret=interpret)

    try:
        devs = jax.devices()
    except RuntimeError:
        devs = []
    if len(devs) >= 2 and not interpret:
        mesh = Mesh(np.array(devs[:2]), ("x",))
        sharded = P("x")
        repl = P(None, None)
        out = jax.shard_map(
            block, mesh=mesh,
            in_specs=(sharded, sharded, sharded, sharded, repl, repl, repl,
                      repl, repl, repl, repl, repl, repl, repl),
            out_specs=P(None, "x", None),
            check_vma=False,
        )(*args)
    else:
        out = block(*args)
    return jnp.transpose(out, (1, 0, 2))
